# baseline (device time: 17653 ns/iter reference)
import jax
import jax.numpy as jnp
from jax import lax
from jax.experimental import pallas as pl
from jax.experimental.pallas import tpu as pltpu

N_DEV = 32
EPS = 1e-5


def kernel(x, t_emb, W_scale, W_shift):
    b, s, c_loc = x.shape
    c_glob = c_loc * N_DEV

    def body(x_hbm, t_hbm, ws_hbm, wsh_hbm, out_ref,
             xv_ref, t_ref, ws_ref, wsh_ref,
             stats_ref, gather_ref, copy_sems, send_sems, recv_sems):
        my = lax.axis_index("i")

        barrier = pltpu.get_barrier_semaphore()
        for off in range(1, N_DEV):
            pl.semaphore_signal(
                barrier, inc=1,
                device_id=((my + off) % N_DEV,),
                device_id_type=pl.DeviceIdType.MESH,
            )

        cp_x = pltpu.make_async_copy(x_hbm, xv_ref, copy_sems.at[0])
        cp_t = pltpu.make_async_copy(t_hbm, t_ref, copy_sems.at[1])
        cp_ws = pltpu.make_async_copy(ws_hbm, ws_ref, copy_sems.at[2])
        cp_wsh = pltpu.make_async_copy(wsh_hbm, wsh_ref, copy_sems.at[3])
        cp_x.start()
        cp_t.start()
        cp_ws.start()
        cp_wsh.start()

        cp_x.wait()
        xv = xv_ref[...]
        stats_ref[0, :, :] = jnp.sum(xv, axis=-1)
        stats_ref[1, :, :] = jnp.sum(xv * xv, axis=-1)

        pl.semaphore_wait(barrier, N_DEV - 1)

        rdmas = []
        for off in range(1, N_DEV):
            rdma = pltpu.make_async_remote_copy(
                src_ref=stats_ref,
                dst_ref=gather_ref.at[off - 1],
                send_sem=send_sems.at[off - 1],
                recv_sem=recv_sems.at[off - 1],
                device_id=((my + off) % N_DEV,),
                device_id_type=pl.DeviceIdType.MESH,
            )
            rdma.start()
            rdmas.append(rdma)

        cp_t.wait()
        cp_ws.wait()
        cp_wsh.wait()
        scale = jnp.dot(t_ref[...], ws_ref[...],
                        preferred_element_type=jnp.float32)
        shift = jnp.dot(t_ref[...], wsh_ref[...],
                        preferred_element_type=jnp.float32)

        for rdma in rdmas:
            rdma.wait_recv()

        tot = stats_ref[...] + jnp.sum(gather_ref[...], axis=0)
        mean = tot[0] * (1.0 / c_glob)
        var = tot[1] * (1.0 / c_glob) - mean * mean
        inv = lax.rsqrt(var + EPS)

        h = (xv - mean[..., None]) * inv[..., None]
        out_ref[...] = h * (1.0 + scale[:, None, :]) + shift[:, None, :]

        for rdma in rdmas:
            rdma.wait_send()

    return pl.pallas_call(
        body,
        out_shape=jax.ShapeDtypeStruct((b, s, c_loc), jnp.float32),
        in_specs=[pl.BlockSpec(memory_space=pl.ANY)] * 4,
        out_specs=pl.BlockSpec(memory_space=pltpu.VMEM),
        scratch_shapes=[
            pltpu.VMEM((b, s, c_loc), jnp.float32),
            pltpu.VMEM(t_emb.shape, jnp.float32),
            pltpu.VMEM(W_scale.shape, jnp.float32),
            pltpu.VMEM(W_shift.shape, jnp.float32),
            pltpu.VMEM((2, b, s), jnp.float32),
            pltpu.VMEM((N_DEV - 1, 2, b, s), jnp.float32),
            pltpu.SemaphoreType.DMA((4,)),
            pltpu.SemaphoreType.DMA((N_DEV - 1,)),
            pltpu.SemaphoreType.DMA((N_DEV - 1,)),
        ],
        compiler_params=pltpu.CompilerParams(collective_id=0),
    )(x, t_emb, W_scale, W_shift)
